# baseline (speedup 1.0000x reference)
import functools

import jax
import jax.numpy as jnp
import numpy as np
from jax import lax
from jax.experimental import pallas as pl
from jax.experimental.pallas import tpu as pltpu
from jax.experimental.pallas import tpu_sc as plsc

N = 10000
E = 160000
HID = 256
NSUB = 16
E16 = E // NSUB
E32 = E // 2 // NSUB
B = 80
B0 = 40
NCH = E16 // B
NCH0 = E32 // B0
NP = 10240
RS = NP // NSUB

BN = 1000
BE = 2000

_EPS = 1e-5


def _pack_pairs(v):
    v3 = v.reshape(v.shape[0] // 2, 2, 128)
    ev = jax.lax.bitcast_convert_type(
        v3[:, 0, :].astype(jnp.bfloat16).astype(jnp.float32), jnp.uint32)
    od = jax.lax.bitcast_convert_type(
        v3[:, 1, :].astype(jnp.bfloat16).astype(jnp.float32), jnp.uint32)
    packed = (ev >> jnp.uint32(16)) | (od & jnp.uint32(0xFFFF0000))
    return jax.lax.bitcast_convert_type(packed, jnp.int32)


def _ln(v, g, b):
    m = jnp.mean(v, axis=-1, keepdims=True)
    var = jnp.mean((v - m) ** 2, axis=-1, keepdims=True)
    return (v - m) / jnp.sqrt(var + _EPS) * g + b


def _node_enc_body(x_ref, w_ref, b_ref, g_ref, bb_ref, o_ref):
    v = jnp.maximum(jnp.dot(x_ref[...], w_ref[...]) + b_ref[...], 0.0)
    o_ref[...] = _ln(v, g_ref[...], bb_ref[...])


def _edge_proj_a_body(ea_ref, we_ref, be_ref, ge_ref, bbe_ref,
                      b1_ref, c1_ref, o1_ref):
    raw = ea_ref[...]
    enc = _ln(jnp.maximum(jnp.dot(raw, we_ref[...]) + be_ref[...], 0.0),
              ge_ref[...], bbe_ref[...])
    o1_ref[...] = _pack_pairs(jnp.dot(enc, b1_ref[0]) + c1_ref[0])


def _edge_proj_b_body(ea_ref, we_ref, be_ref, ge_ref, bbe_ref,
                      b2_ref, c2_ref, cc_ref, ccb_ref, o2_ref, oc_ref):
    raw = ea_ref[...]
    enc = _ln(jnp.maximum(jnp.dot(raw, we_ref[...]) + be_ref[...], 0.0),
              ge_ref[...], bbe_ref[...])
    o2_ref[...] = _pack_pairs(jnp.dot(enc, b2_ref[0]) + c2_ref[0])
    oc_ref[...] = _pack_pairs(jnp.dot(raw, cc_ref[0]) + ccb_ref[0])


def _ha_body(h_ref, a_ref, o_ref):
    o_ref[...] = jnp.dot(h_ref[...], a_ref[0])


def _upd_body(h_ref, m0_ref, m1_ref, ca_ref, cb_ref, wu0_ref, wu1_ref,
              wu2_ref, bu_ref, g_ref, b_ref, o_ref):
    h = h_ref[...]
    cnt = ca_ref[:, 0:1] + cb_ref[:, 0:1]
    inv = 1.0 / jnp.maximum(cnt, 1.0)
    u = (jnp.dot(h, wu0_ref[...]) + jnp.dot(m0_ref[...] * inv, wu1_ref[...])
         + jnp.dot(m1_ref[...] * inv, wu2_ref[...]) + bu_ref[...])
    u = _ln(u, g_ref[...], b_ref[...])
    o_ref[...] = jnp.maximum(u + h, 0.0)


def _pred_body(zl_ref, zh_ref, w2_ref, b2_ref, w3_ref, b3_ref, o_ref):
    z = jnp.maximum(jnp.concatenate([zl_ref[...], zh_ref[...]], axis=1), 0.0)
    z = jnp.maximum(jnp.dot(z, w2_ref[...]) + b2_ref[...], 0.0)
    o_ref[...] = jax.nn.sigmoid(jnp.dot(z, w3_ref[...]) + b3_ref[...])


@functools.cache
def _sc_kernels():
  mesh = plsc.VectorSubcoreMesh(
      core_axis_name="c", subcore_axis_name="s", num_cores=2, num_subcores=NSUB)

  @functools.partial(
    pl.kernel,
    out_type=(jax.ShapeDtypeStruct((NP, 128), jnp.float32),
              jax.ShapeDtypeStruct((NP, 128), jnp.float32)),
    mesh=mesh,
    scratch_types=[
        pltpu.VMEM((B0,), jnp.int32),
        pltpu.VMEM((B0, 128), jnp.float32),
        pltpu.VMEM_SHARED((NP, 128), jnp.float32),
    ],
  )
  def _s0(dst, ones, zeros, out0, out1, dstv, onesv, shared):
    c = lax.axis_index("c")
    s = lax.axis_index("s")
    pltpu.sync_copy(zeros.at[pl.ds(s * RS, RS)], shared.at[pl.ds(s * RS, RS)])
    pltpu.sync_copy(ones, onesv)
    plsc.subcore_barrier()
    base = c * (E // 2) + s * E32

    def chunk(j, carry):
        pltpu.sync_copy(dst.at[pl.ds(base + j * B0, B0)], dstv)
        pltpu.sync_copy(onesv, shared.at[dstv], add=True)
        return carry

    lax.fori_loop(0, NCH0, chunk, 0)
    plsc.subcore_barrier()

    @pl.when(c == 0)
    def _():
        pltpu.sync_copy(shared.at[pl.ds(s * RS, RS)], out0.at[pl.ds(s * RS, RS)])

    @pl.when(c == 1)
    def _():
        pltpu.sync_copy(shared.at[pl.ds(s * RS, RS)], out1.at[pl.ds(s * RS, RS)])

  @functools.partial(
    pl.kernel,
    out_type=(jax.ShapeDtypeStruct((NP, 128), jnp.float32),
              jax.ShapeDtypeStruct((NP, 128), jnp.float32)),
    mesh=mesh,
    scratch_types=[
        pltpu.VMEM((2, B), jnp.int32),
        pltpu.VMEM((2, B), jnp.int32),
        pltpu.VMEM((2, B // 2, 128), jnp.int32),
        pltpu.VMEM((2, B, 128), jnp.float32),
        pltpu.VMEM_SHARED((NP, 128), jnp.float32),
        pltpu.SemaphoreType.DMA,
        pltpu.SemaphoreType.DMA,
        pltpu.SemaphoreType.DMA,
        pltpu.SemaphoreType.DMA,
        pltpu.SemaphoreType.DMA,
        pltpu.SemaphoreType.DMA,
        pltpu.SemaphoreType.DMA,
        pltpu.SemaphoreType.DMA,
    ],
  )
  def _s1(hA, eaB, srcx4, dst3, zeros, out0, out1, srcv, dstv, ebuf, gbuf,
          shared, se0, se1, sg0, sg1, si0, si1, sd0, sd1):
    c = lax.axis_index("c")
    s = lax.axis_index("s")
    se = (se0, se1)
    sg = (sg0, sg1)
    si = (si0, si1)
    sd = (sd0, sd1)
    w = c * NSUB + s
    pltpu.sync_copy(zeros.at[pl.ds(s * RS, RS)], shared.at[pl.ds(s * RS, RS)])
    plsc.subcore_barrier()
    ebase = c * E + s * E16

    def src_copy(j, slot):
        return pltpu.make_async_copy(srcx4.at[w, j], srcv.at[slot], si[slot])

    def dst_copy(j, slot):
        return pltpu.make_async_copy(dst3.at[s, j], dstv.at[slot], sd[slot])

    ebase2 = c * (E // 2) + s * (E16 // 2)

    def eaB_copy(j, slot):
        return pltpu.make_async_copy(
            eaB.at[pl.ds(ebase2 + j * (B // 2), B // 2)], ebuf.at[slot],
            se[slot])

    def gather_copy(slot):
        return pltpu.make_async_copy(
            hA.at[srcv.at[slot]], gbuf.at[slot], sg[slot])

    pltpu.sync_copy(srcx4.at[w, 0], srcv.at[0])
    pltpu.sync_copy(dst3.at[s, 0], dstv.at[0])
    src_copy(1, 1).start()
    dst_copy(1, 1).start()
    gather_copy(0).start()
    eaB_copy(0, 0).start()
    eaB_copy(1, 1).start()

    def body(j, slot):
        nslot = 1 - slot
        gather_copy(slot).wait()
        eaB_copy(j, slot).wait()

        def row(t, carry2):
            for k in range(128 // 16):
                sl = pl.ds(k * 16, 16)
                u = ebuf[slot, t, sl]
                lo = jax.lax.bitcast_convert_type(u << 16, jnp.float32)
                hi = jax.lax.bitcast_convert_type(u & jnp.int32(-65536),
                                                  jnp.float32)
                gbuf[slot, 2 * t, sl] = jnp.maximum(gbuf[slot, 2 * t, sl] + lo,
                                                    0.0)
                gbuf[slot, 2 * t + 1, sl] = jnp.maximum(
                    gbuf[slot, 2 * t + 1, sl] + hi, 0.0)
            return carry2

        lax.fori_loop(0, B // 2, row, 0)

        @pl.when(j + 1 < NCH)
        def _():
            src_copy(j + 1, nslot).wait()
            gather_copy(nslot).start()

        @pl.when(j > 0)
        def _():
            dst_copy(j, slot).wait()

        pltpu.sync_copy(gbuf.at[slot], shared.at[dstv.at[slot]], add=True)

        @pl.when(j + 2 < NCH)
        def _():
            src_copy(j + 2, slot).start()
            dst_copy(j + 2, slot).start()
            eaB_copy(j + 2, slot).start()

    def chunk(j, carry):
        @pl.when(j % 2 == 0)
        def _():
            body(j, 0)

        @pl.when(j % 2 == 1)
        def _():
            body(j, 1)

        return carry

    lax.fori_loop(0, NCH, chunk, 0)
    plsc.subcore_barrier()

    @pl.when(c == 0)
    def _():
        pltpu.sync_copy(shared.at[pl.ds(s * RS, RS)], out0.at[pl.ds(s * RS, RS)])

    @pl.when(c == 1)
    def _():
        pltpu.sync_copy(shared.at[pl.ds(s * RS, RS)], out1.at[pl.ds(s * RS, RS)])

  @functools.partial(
    pl.kernel,
    out_type=jax.ShapeDtypeStruct((2 * E, 128), jnp.float32),
    mesh=mesh,
    scratch_types=[
        pltpu.VMEM((NCH, B), jnp.int32),
        pltpu.VMEM((NCH, B), jnp.int32),
        pltpu.VMEM((2, B // 2, 128), jnp.int32),
        pltpu.VMEM((2, B, 128), jnp.float32),
        pltpu.VMEM((2, B, 128), jnp.float32),
        pltpu.SemaphoreType.DMA,
        pltpu.SemaphoreType.DMA,
        pltpu.SemaphoreType.DMA,
        pltpu.SemaphoreType.DMA,
        pltpu.SemaphoreType.DMA,
        pltpu.SemaphoreType.DMA,
    ],
  )
  def _s2(T, eC, srcx4, dstx4, out, src_t, dst_t, ebuf, g1, g2,
          se0, se1, sa0, sa1, sb0, sb1):
    c = lax.axis_index("c")
    s = lax.axis_index("s")
    se = (se0, se1)
    sa = (sa0, sa1)
    sb = (sb0, sb1)
    pltpu.sync_copy(srcx4.at[c * NSUB + s], src_t)
    pltpu.sync_copy(dstx4.at[c * NSUB + s], dst_t)
    ebase = c * E + s * E16

    ebase2 = c * (E // 2) + s * (E16 // 2)

    def eC_copy(j, slot):
        return pltpu.make_async_copy(
            eC.at[pl.ds(ebase2 + j * (B // 2), B // 2)], ebuf.at[slot],
            se[slot])

    def ga_copy(j, slot):
        return pltpu.make_async_copy(T.at[src_t.at[j]], g1.at[slot], sa[slot])

    def gb_copy(j, slot):
        return pltpu.make_async_copy(T.at[dst_t.at[j]], g2.at[slot], sb[slot])

    for slot in (0, 1):
        eC_copy(slot, slot).start()
        ga_copy(slot, slot).start()
        gb_copy(slot, slot).start()

    def body(j, slot):
        eC_copy(j, slot).wait()
        ga_copy(j, slot).wait()
        gb_copy(j, slot).wait()

        def row(t, carry2):
            for k in range(128 // 16):
                sl = pl.ds(k * 16, 16)
                u = ebuf[slot, t, sl]
                lo = jax.lax.bitcast_convert_type(u << 16, jnp.float32)
                hi = jax.lax.bitcast_convert_type(u & jnp.int32(-65536),
                                                  jnp.float32)
                g1[slot, 2 * t, sl] = (g1[slot, 2 * t, sl]
                                       + g2[slot, 2 * t, sl] + lo)
                g1[slot, 2 * t + 1, sl] = (g1[slot, 2 * t + 1, sl]
                                           + g2[slot, 2 * t + 1, sl] + hi)
            return carry2

        lax.fori_loop(0, B // 2, row, 0)
        pltpu.sync_copy(g1.at[slot], out.at[pl.ds(ebase + j * B, B)])

        @pl.when(j + 2 < NCH)
        def _():
            eC_copy(j + 2, slot).start()
            ga_copy(j + 2, slot).start()
            gb_copy(j + 2, slot).start()

    def chunk(j, carry):
        @pl.when(j % 2 == 0)
        def _():
            body(j, 0)

        @pl.when(j % 2 == 1)
        def _():
            body(j, 1)

        return carry

    lax.fori_loop(0, NCH, chunk, 0)

  return _s0, _s1, _s2


def _row2(m):
    return m.reshape(1, -1)


def _split_stack(w):
    return w.reshape(w.shape[0], 2, 128).transpose(1, 0, 2)


def kernel(x, edge_index, edge_attr, params):
    src = edge_index[0]
    dst = edge_index[1]
    nE = E // BE
    nN = N // BN

    pn = params["node_enc"]
    pe = params["edge_enc"]
    pp = params["pred"]

    h = pl.pallas_call(
        _node_enc_body,
        grid=(nN,),
        in_specs=[
            pl.BlockSpec((BN, 128), lambda i: (i, 0)),
            pl.BlockSpec((128, 256), lambda i: (0, 0)),
            pl.BlockSpec((1, 256), lambda i: (0, 0)),
            pl.BlockSpec((1, 256), lambda i: (0, 0)),
            pl.BlockSpec((1, 256), lambda i: (0, 0)),
        ],
        out_specs=pl.BlockSpec((BN, 256), lambda i: (i, 0)),
        out_shape=jax.ShapeDtypeStruct((N, 256), jnp.float32),
    )(x, pn["lin"]["w"], _row2(pn["lin"]["b"]), _row2(pn["ln_g"]),
      _row2(pn["ln_b"]))

    Bst = [_split_stack(cp["msg"]["w"][256:]) for cp in params["convs"]]
    bst = [cp["msg"]["b"].reshape(2, 1, 128) for cp in params["convs"]]
    W1 = pp["l1"]["w"]
    Cst = W1[512:].reshape(16, 2, 128).transpose(1, 0, 2)
    cbs = pp["l1"]["b"].reshape(2, 1, 128)

    _enc_specs = [
        pl.BlockSpec((BE, 16), lambda c, i: (i, 0)),
        pl.BlockSpec((16, 256), lambda c, i: (0, 0)),
        pl.BlockSpec((1, 256), lambda c, i: (0, 0)),
        pl.BlockSpec((1, 256), lambda c, i: (0, 0)),
        pl.BlockSpec((1, 256), lambda c, i: (0, 0)),
    ]
    _enc_args = (edge_attr, pe["lin"]["w"], _row2(pe["lin"]["b"]),
                 _row2(pe["ln_g"]), _row2(pe["ln_b"]))
    _eout = pl.BlockSpec((BE // 2, 128), lambda c, i: (c * (E // BE) + i, 0))

    srcx4 = jnp.concatenate([src, src + N]).reshape(2 * NSUB, NCH, B)
    dstx4 = jnp.concatenate([dst + 2 * N, dst + 3 * N]).reshape(2 * NSUB, NCH, B)
    dst3 = dst.reshape(NSUB, NCH, B)
    zeros = jnp.zeros((NP, 128), jnp.float32)
    ones = jnp.ones((B0, 128), jnp.float32)
    _s0, _s1, _s2 = _sc_kernels()

    cnta, cntb = _s0(dst, ones, zeros)

    eaB1 = pl.pallas_call(
        _edge_proj_a_body,
        grid=(2, nE),
        in_specs=_enc_specs + [
            pl.BlockSpec((1, 256, 128), lambda c, i: (c, 0, 0)),
            pl.BlockSpec((1, 1, 128), lambda c, i: (c, 0, 0)),
        ],
        out_specs=_eout,
        out_shape=jax.ShapeDtypeStruct((E, 128), jnp.int32),
    )(*_enc_args, Bst[0], bst[0])

    eaB2 = eC = None
    for li, cp in enumerate(params["convs"]):
        Ast = _split_stack(cp["msg"]["w"][:256])
        hA = pl.pallas_call(
            _ha_body,
            grid=(2, nN),
            in_specs=[
                pl.BlockSpec((BN, 256), lambda c, i: (i, 0)),
                pl.BlockSpec((1, 256, 128), lambda c, i: (c, 0, 0)),
            ],
            out_specs=pl.BlockSpec((BN, 128), lambda c, i: (c * (N // BN) + i, 0)),
            out_shape=jax.ShapeDtypeStruct((2 * N, 128), jnp.float32),
        )(h, Ast)

        msum0, msum1 = _s1(hA, eaB1 if li == 0 else eaB2, srcx4, dst3, zeros)

        if li == 0:
            eaB2, eC = pl.pallas_call(
                _edge_proj_b_body,
                grid=(2, nE),
                in_specs=_enc_specs + [
                    pl.BlockSpec((1, 256, 128), lambda c, i: (c, 0, 0)),
                    pl.BlockSpec((1, 1, 128), lambda c, i: (c, 0, 0)),
                    pl.BlockSpec((1, 16, 128), lambda c, i: (c, 0, 0)),
                    pl.BlockSpec((1, 1, 128), lambda c, i: (c, 0, 0)),
                ],
                out_specs=[_eout, _eout],
                out_shape=[
                    jax.ShapeDtypeStruct((E, 128), jnp.int32),
                    jax.ShapeDtypeStruct((E, 128), jnp.int32),
                ],
            )(*_enc_args, Bst[1], bst[1], Cst, cbs)

        Wu = cp["upd"]["w"]
        h = pl.pallas_call(
            _upd_body,
            grid=(nN,),
            in_specs=[
                pl.BlockSpec((BN, 256), lambda i: (i, 0)),
                pl.BlockSpec((BN, 128), lambda i: (i, 0)),
                pl.BlockSpec((BN, 128), lambda i: (i, 0)),
                pl.BlockSpec((BN, 128), lambda i: (i, 0)),
                pl.BlockSpec((BN, 128), lambda i: (i, 0)),
                pl.BlockSpec((256, 256), lambda i: (0, 0)),
                pl.BlockSpec((128, 256), lambda i: (0, 0)),
                pl.BlockSpec((128, 256), lambda i: (0, 0)),
                pl.BlockSpec((1, 256), lambda i: (0, 0)),
                pl.BlockSpec((1, 256), lambda i: (0, 0)),
                pl.BlockSpec((1, 256), lambda i: (0, 0)),
            ],
            out_specs=pl.BlockSpec((BN, 256), lambda i: (i, 0)),
            out_shape=jax.ShapeDtypeStruct((N, 256), jnp.float32),
        )(h, msum0, msum1, cnta, cntb, Wu[:256], Wu[256:384], Wu[384:512],
          _row2(cp["upd"]["b"]), _row2(cp["ln_g"]), _row2(cp["ln_b"]))

    Wt = jnp.stack([
        W1[:256, :128], W1[:256, 128:256], W1[256:512, :128], W1[256:512, 128:256],
    ])
    T = pl.pallas_call(
        _ha_body,
        grid=(4, nN),
        in_specs=[
            pl.BlockSpec((BN, 256), lambda c, i: (i, 0)),
            pl.BlockSpec((1, 256, 128), lambda c, i: (c, 0, 0)),
        ],
        out_specs=pl.BlockSpec((BN, 128), lambda c, i: (c * (N // BN) + i, 0)),
        out_shape=jax.ShapeDtypeStruct((4 * N, 128), jnp.float32),
    )(h, Wt)

    z1pre = _s2(T, eC, srcx4, dstx4)

    out = pl.pallas_call(
        _pred_body,
        grid=(nE,),
        in_specs=[
            pl.BlockSpec((BE, 128), lambda i: (i, 0)),
            pl.BlockSpec((BE, 128), lambda i: (E // BE + i, 0)),
            pl.BlockSpec((256, 128), lambda i: (0, 0)),
            pl.BlockSpec((1, 128), lambda i: (0, 0)),
            pl.BlockSpec((128, 1), lambda i: (0, 0)),
            pl.BlockSpec((1, 1), lambda i: (0, 0)),
        ],
        out_specs=pl.BlockSpec((BE, 1), lambda i: (i, 0)),
        out_shape=jax.ShapeDtypeStruct((E, 1), jnp.float32),
    )(z1pre, z1pre, pp["l2"]["w"], _row2(pp["l2"]["b"]), pp["l3"]["w"],
      _row2(pp["l3"]["b"]))

    return out


# speedup vs baseline: 1.4555x; 1.4555x over previous
import functools

import jax
import jax.numpy as jnp
import numpy as np
from jax import lax
from jax.experimental import pallas as pl
from jax.experimental.pallas import tpu as pltpu
from jax.experimental.pallas import tpu_sc as plsc

N = 10000
E = 160000
HID = 256
NSUB = 16
E16 = E // NSUB
E32 = E // 2 // NSUB
B = 80
B0 = 40
NCH = E16 // B
NCH0 = E32 // B0
NP = 10240
RS = NP // NSUB

BN = 1000
BE = 2000

_EPS = 1e-5


def _pack_halves(v):
    lo = jax.lax.bitcast_convert_type(
        v[:, :128].astype(jnp.bfloat16).astype(jnp.float32), jnp.uint32)
    hi = jax.lax.bitcast_convert_type(
        v[:, 128:].astype(jnp.bfloat16).astype(jnp.float32), jnp.uint32)
    packed = (lo >> jnp.uint32(16)) | (hi & jnp.uint32(0xFFFF0000))
    return jax.lax.bitcast_convert_type(packed, jnp.int32)


def _ln(v, g, b):
    m = jnp.mean(v, axis=-1, keepdims=True)
    var = jnp.mean((v - m) ** 2, axis=-1, keepdims=True)
    return (v - m) / jnp.sqrt(var + _EPS) * g + b


def _node_enc_body(x_ref, w_ref, b_ref, g_ref, bb_ref, o_ref):
    v = jnp.maximum(jnp.dot(x_ref[...], w_ref[...]) + b_ref[...], 0.0)
    o_ref[...] = _ln(v, g_ref[...], bb_ref[...])


def _edge_proj_a_body(ea_ref, we_ref, be_ref, ge_ref, bbe_ref,
                      b1_ref, c1_ref, o1_ref):
    raw = ea_ref[...]
    enc = _ln(jnp.maximum(jnp.dot(raw, we_ref[...]) + be_ref[...], 0.0),
              ge_ref[...], bbe_ref[...])
    o1_ref[...] = _pack_halves(jnp.dot(enc, b1_ref[...]) + c1_ref[...])


def _edge_proj_b_body(ea_ref, we_ref, be_ref, ge_ref, bbe_ref,
                      b2_ref, c2_ref, cc_ref, ccb_ref, o2_ref, oc_ref):
    raw = ea_ref[...]
    enc = _ln(jnp.maximum(jnp.dot(raw, we_ref[...]) + be_ref[...], 0.0),
              ge_ref[...], bbe_ref[...])
    o2_ref[...] = _pack_halves(jnp.dot(enc, b2_ref[...]) + c2_ref[...])
    oc_ref[...] = _pack_halves(jnp.dot(raw, cc_ref[...]) + ccb_ref[...])


def _ha_body(h_ref, a_ref, o_ref):
    o_ref[...] = jnp.dot(h_ref[...], a_ref[0])


def _upd_body(h_ref, m0_ref, m1_ref, ca_ref, cb_ref, wu0_ref, wu1_ref,
              wu2_ref, bu_ref, g_ref, b_ref, o_ref):
    h = h_ref[...]
    cnt = ca_ref[:, 0:1] + cb_ref[:, 0:1]
    inv = 1.0 / jnp.maximum(cnt, 1.0)
    u = (jnp.dot(h, wu0_ref[...]) + jnp.dot(m0_ref[...] * inv, wu1_ref[...])
         + jnp.dot(m1_ref[...] * inv, wu2_ref[...]) + bu_ref[...])
    u = _ln(u, g_ref[...], b_ref[...])
    o_ref[...] = jnp.maximum(u + h, 0.0)


def _pred_body(zl_ref, zh_ref, w2_ref, b2_ref, w3_ref, b3_ref, o_ref):
    z = jnp.maximum(jnp.concatenate([zl_ref[...], zh_ref[...]], axis=1), 0.0)
    z = jnp.maximum(jnp.dot(z, w2_ref[...]) + b2_ref[...], 0.0)
    o_ref[...] = jax.nn.sigmoid(jnp.dot(z, w3_ref[...]) + b3_ref[...])


@functools.cache
def _sc_kernels():
  mesh = plsc.VectorSubcoreMesh(
      core_axis_name="c", subcore_axis_name="s", num_cores=2, num_subcores=NSUB)

  @functools.partial(
    pl.kernel,
    out_type=(jax.ShapeDtypeStruct((NP, 128), jnp.float32),
              jax.ShapeDtypeStruct((NP, 128), jnp.float32)),
    mesh=mesh,
    scratch_types=[
        pltpu.VMEM((B0,), jnp.int32),
        pltpu.VMEM((B0, 128), jnp.float32),
        pltpu.VMEM_SHARED((NP, 128), jnp.float32),
    ],
  )
  def _s0(dst, ones, zeros, out0, out1, dstv, onesv, shared):
    c = lax.axis_index("c")
    s = lax.axis_index("s")
    pltpu.sync_copy(zeros.at[pl.ds(s * RS, RS)], shared.at[pl.ds(s * RS, RS)])
    pltpu.sync_copy(ones, onesv)
    plsc.subcore_barrier()
    base = c * (E // 2) + s * E32

    def chunk(j, carry):
        pltpu.sync_copy(dst.at[pl.ds(base + j * B0, B0)], dstv)
        pltpu.sync_copy(onesv, shared.at[dstv], add=True)
        return carry

    lax.fori_loop(0, NCH0, chunk, 0)
    plsc.subcore_barrier()

    @pl.when(c == 0)
    def _():
        pltpu.sync_copy(shared.at[pl.ds(s * RS, RS)], out0.at[pl.ds(s * RS, RS)])

    @pl.when(c == 1)
    def _():
        pltpu.sync_copy(shared.at[pl.ds(s * RS, RS)], out1.at[pl.ds(s * RS, RS)])

  @functools.partial(
    pl.kernel,
    out_type=(jax.ShapeDtypeStruct((NP, 128), jnp.float32),
              jax.ShapeDtypeStruct((NP, 128), jnp.float32)),
    mesh=mesh,
    scratch_types=[
        pltpu.VMEM((2, B), jnp.int32),
        pltpu.VMEM((2, B), jnp.int32),
        pltpu.VMEM((2, B, 128), jnp.int32),
        pltpu.VMEM((2, B, 128), jnp.float32),
        pltpu.VMEM_SHARED((NP, 128), jnp.float32),
        pltpu.SemaphoreType.DMA,
        pltpu.SemaphoreType.DMA,
        pltpu.SemaphoreType.DMA,
        pltpu.SemaphoreType.DMA,
        pltpu.SemaphoreType.DMA,
        pltpu.SemaphoreType.DMA,
        pltpu.SemaphoreType.DMA,
        pltpu.SemaphoreType.DMA,
    ],
  )
  def _s1(hA, eaB, srcx4, dst3, zeros, out0, out1, srcv, dstv, ebuf, gbuf,
          shared, se0, se1, sg0, sg1, si0, si1, sd0, sd1):
    c = lax.axis_index("c")
    s = lax.axis_index("s")
    se = (se0, se1)
    sg = (sg0, sg1)
    si = (si0, si1)
    sd = (sd0, sd1)
    w = c * NSUB + s
    pltpu.sync_copy(zeros.at[pl.ds(s * RS, RS)], shared.at[pl.ds(s * RS, RS)])
    plsc.subcore_barrier()
    ebase = c * E + s * E16

    def src_copy(j, slot):
        return pltpu.make_async_copy(srcx4.at[w, j], srcv.at[slot], si[slot])

    def dst_copy(j, slot):
        return pltpu.make_async_copy(dst3.at[s, j], dstv.at[slot], sd[slot])

    sbase = s * E16
    shift = 16 * (1 - c)

    def eaB_copy(j, slot):
        return pltpu.make_async_copy(
            eaB.at[pl.ds(sbase + j * B, B)], ebuf.at[slot], se[slot])

    def gather_copy(slot):
        return pltpu.make_async_copy(
            hA.at[srcv.at[slot]], gbuf.at[slot], sg[slot])

    pltpu.sync_copy(srcx4.at[w, 0], srcv.at[0])
    pltpu.sync_copy(dst3.at[s, 0], dstv.at[0])
    src_copy(1, 1).start()
    dst_copy(1, 1).start()
    gather_copy(0).start()
    eaB_copy(0, 0).start()
    eaB_copy(1, 1).start()

    def body(j, slot):
        nslot = 1 - slot
        gather_copy(slot).wait()
        eaB_copy(j, slot).wait()

        def row(r, carry2):
            for k in range(128 // 16):
                sl = pl.ds(k * 16, 16)
                u = ebuf[slot, r, sl]
                v = jax.lax.bitcast_convert_type(
                    (u << shift) & jnp.int32(-65536), jnp.float32)
                gbuf[slot, r, sl] = jnp.maximum(gbuf[slot, r, sl] + v, 0.0)
            return carry2

        lax.fori_loop(0, B, row, 0)

        @pl.when(j + 1 < NCH)
        def _():
            src_copy(j + 1, nslot).wait()
            gather_copy(nslot).start()

        @pl.when(j > 0)
        def _():
            dst_copy(j, slot).wait()

        pltpu.sync_copy(gbuf.at[slot], shared.at[dstv.at[slot]], add=True)

        @pl.when(j + 2 < NCH)
        def _():
            src_copy(j + 2, slot).start()
            dst_copy(j + 2, slot).start()
            eaB_copy(j + 2, slot).start()

    def chunk(j, carry):
        @pl.when(j % 2 == 0)
        def _():
            body(j, 0)

        @pl.when(j % 2 == 1)
        def _():
            body(j, 1)

        return carry

    lax.fori_loop(0, NCH, chunk, 0)
    plsc.subcore_barrier()

    @pl.when(c == 0)
    def _():
        pltpu.sync_copy(shared.at[pl.ds(s * RS, RS)], out0.at[pl.ds(s * RS, RS)])

    @pl.when(c == 1)
    def _():
        pltpu.sync_copy(shared.at[pl.ds(s * RS, RS)], out1.at[pl.ds(s * RS, RS)])

  @functools.partial(
    pl.kernel,
    out_type=jax.ShapeDtypeStruct((2 * E, 128), jnp.float32),
    mesh=mesh,
    scratch_types=[
        pltpu.VMEM((NCH, B), jnp.int32),
        pltpu.VMEM((NCH, B), jnp.int32),
        pltpu.VMEM((2, B, 128), jnp.int32),
        pltpu.VMEM((2, B, 128), jnp.float32),
        pltpu.VMEM((2, B, 128), jnp.float32),
        pltpu.SemaphoreType.DMA,
        pltpu.SemaphoreType.DMA,
        pltpu.SemaphoreType.DMA,
        pltpu.SemaphoreType.DMA,
        pltpu.SemaphoreType.DMA,
        pltpu.SemaphoreType.DMA,
    ],
  )
  def _s2(T, eC, srcx4, dstx4, out, src_t, dst_t, ebuf, g1, g2,
          se0, se1, sa0, sa1, sb0, sb1):
    c = lax.axis_index("c")
    s = lax.axis_index("s")
    se = (se0, se1)
    sa = (sa0, sa1)
    sb = (sb0, sb1)
    pltpu.sync_copy(srcx4.at[c * NSUB + s], src_t)
    pltpu.sync_copy(dstx4.at[c * NSUB + s], dst_t)
    ebase = c * E + s * E16

    sbase = s * E16
    shift = 16 * (1 - c)

    def eC_copy(j, slot):
        return pltpu.make_async_copy(
            eC.at[pl.ds(sbase + j * B, B)], ebuf.at[slot], se[slot])

    def ga_copy(j, slot):
        return pltpu.make_async_copy(T.at[src_t.at[j]], g1.at[slot], sa[slot])

    def gb_copy(j, slot):
        return pltpu.make_async_copy(T.at[dst_t.at[j]], g2.at[slot], sb[slot])

    for slot in (0, 1):
        eC_copy(slot, slot).start()
        ga_copy(slot, slot).start()
        gb_copy(slot, slot).start()

    def body(j, slot):
        eC_copy(j, slot).wait()
        ga_copy(j, slot).wait()
        gb_copy(j, slot).wait()

        def row(r, carry2):
            for k in range(128 // 16):
                sl = pl.ds(k * 16, 16)
                u = ebuf[slot, r, sl]
                v = jax.lax.bitcast_convert_type(
                    (u << shift) & jnp.int32(-65536), jnp.float32)
                g1[slot, r, sl] = g1[slot, r, sl] + g2[slot, r, sl] + v
            return carry2

        lax.fori_loop(0, B, row, 0)
        pltpu.sync_copy(g1.at[slot], out.at[pl.ds(ebase + j * B, B)])

        @pl.when(j + 2 < NCH)
        def _():
            eC_copy(j + 2, slot).start()
            ga_copy(j + 2, slot).start()
            gb_copy(j + 2, slot).start()

    def chunk(j, carry):
        @pl.when(j % 2 == 0)
        def _():
            body(j, 0)

        @pl.when(j % 2 == 1)
        def _():
            body(j, 1)

        return carry

    lax.fori_loop(0, NCH, chunk, 0)

  return _s0, _s1, _s2


def _row2(m):
    return m.reshape(1, -1)


def _split_stack(w):
    return w.reshape(w.shape[0], 2, 128).transpose(1, 0, 2)


def kernel(x, edge_index, edge_attr, params):
    src = edge_index[0]
    dst = edge_index[1]
    nE = E // BE
    nN = N // BN

    pn = params["node_enc"]
    pe = params["edge_enc"]
    pp = params["pred"]

    h = pl.pallas_call(
        _node_enc_body,
        grid=(nN,),
        in_specs=[
            pl.BlockSpec((BN, 128), lambda i: (i, 0)),
            pl.BlockSpec((128, 256), lambda i: (0, 0)),
            pl.BlockSpec((1, 256), lambda i: (0, 0)),
            pl.BlockSpec((1, 256), lambda i: (0, 0)),
            pl.BlockSpec((1, 256), lambda i: (0, 0)),
        ],
        out_specs=pl.BlockSpec((BN, 256), lambda i: (i, 0)),
        out_shape=jax.ShapeDtypeStruct((N, 256), jnp.float32),
    )(x, pn["lin"]["w"], _row2(pn["lin"]["b"]), _row2(pn["ln_g"]),
      _row2(pn["ln_b"]))

    Bst = [cp["msg"]["w"][256:] for cp in params["convs"]]
    bst = [_row2(cp["msg"]["b"]) for cp in params["convs"]]
    W1 = pp["l1"]["w"]
    Cst = W1[512:]
    cbs = _row2(pp["l1"]["b"])

    _enc_specs = [
        pl.BlockSpec((BE, 16), lambda i: (i, 0)),
        pl.BlockSpec((16, 256), lambda i: (0, 0)),
        pl.BlockSpec((1, 256), lambda i: (0, 0)),
        pl.BlockSpec((1, 256), lambda i: (0, 0)),
        pl.BlockSpec((1, 256), lambda i: (0, 0)),
    ]
    _enc_args = (edge_attr, pe["lin"]["w"], _row2(pe["lin"]["b"]),
                 _row2(pe["ln_g"]), _row2(pe["ln_b"]))
    _eout = pl.BlockSpec((BE, 128), lambda i: (i, 0))

    srcx4 = jnp.concatenate([src, src + N]).reshape(2 * NSUB, NCH, B)
    dstx4 = jnp.concatenate([dst + 2 * N, dst + 3 * N]).reshape(2 * NSUB, NCH, B)
    dst3 = dst.reshape(NSUB, NCH, B)
    zeros = jnp.zeros((NP, 128), jnp.float32)
    ones = jnp.ones((B0, 128), jnp.float32)
    _s0, _s1, _s2 = _sc_kernels()

    cnta, cntb = _s0(dst, ones, zeros)

    eaB1 = pl.pallas_call(
        _edge_proj_a_body,
        grid=(nE,),
        in_specs=_enc_specs + [
            pl.BlockSpec((256, 256), lambda i: (0, 0)),
            pl.BlockSpec((1, 256), lambda i: (0, 0)),
        ],
        out_specs=_eout,
        out_shape=jax.ShapeDtypeStruct((E, 128), jnp.int32),
    )(*_enc_args, Bst[0], bst[0])

    eaB2 = eC = None
    for li, cp in enumerate(params["convs"]):
        Ast = _split_stack(cp["msg"]["w"][:256])
        hA = pl.pallas_call(
            _ha_body,
            grid=(2, nN),
            in_specs=[
                pl.BlockSpec((BN, 256), lambda c, i: (i, 0)),
                pl.BlockSpec((1, 256, 128), lambda c, i: (c, 0, 0)),
            ],
            out_specs=pl.BlockSpec((BN, 128), lambda c, i: (c * (N // BN) + i, 0)),
            out_shape=jax.ShapeDtypeStruct((2 * N, 128), jnp.float32),
        )(h, Ast)

        msum0, msum1 = _s1(hA, eaB1 if li == 0 else eaB2, srcx4, dst3, zeros)

        if li == 0:
            eaB2, eC = pl.pallas_call(
                _edge_proj_b_body,
                grid=(nE,),
                in_specs=_enc_specs + [
                    pl.BlockSpec((256, 256), lambda i: (0, 0)),
                    pl.BlockSpec((1, 256), lambda i: (0, 0)),
                    pl.BlockSpec((16, 256), lambda i: (0, 0)),
                    pl.BlockSpec((1, 256), lambda i: (0, 0)),
                ],
                out_specs=[_eout, _eout],
                out_shape=[
                    jax.ShapeDtypeStruct((E, 128), jnp.int32),
                    jax.ShapeDtypeStruct((E, 128), jnp.int32),
                ],
            )(*_enc_args, Bst[1], bst[1], Cst, cbs)

        Wu = cp["upd"]["w"]
        h = pl.pallas_call(
            _upd_body,
            grid=(nN,),
            in_specs=[
                pl.BlockSpec((BN, 256), lambda i: (i, 0)),
                pl.BlockSpec((BN, 128), lambda i: (i, 0)),
                pl.BlockSpec((BN, 128), lambda i: (i, 0)),
                pl.BlockSpec((BN, 128), lambda i: (i, 0)),
                pl.BlockSpec((BN, 128), lambda i: (i, 0)),
                pl.BlockSpec((256, 256), lambda i: (0, 0)),
                pl.BlockSpec((128, 256), lambda i: (0, 0)),
                pl.BlockSpec((128, 256), lambda i: (0, 0)),
                pl.BlockSpec((1, 256), lambda i: (0, 0)),
                pl.BlockSpec((1, 256), lambda i: (0, 0)),
                pl.BlockSpec((1, 256), lambda i: (0, 0)),
            ],
            out_specs=pl.BlockSpec((BN, 256), lambda i: (i, 0)),
            out_shape=jax.ShapeDtypeStruct((N, 256), jnp.float32),
        )(h, msum0, msum1, cnta, cntb, Wu[:256], Wu[256:384], Wu[384:512],
          _row2(cp["upd"]["b"]), _row2(cp["ln_g"]), _row2(cp["ln_b"]))

    Wt = jnp.stack([
        W1[:256, :128], W1[:256, 128:256], W1[256:512, :128], W1[256:512, 128:256],
    ])
    T = pl.pallas_call(
        _ha_body,
        grid=(4, nN),
        in_specs=[
            pl.BlockSpec((BN, 256), lambda c, i: (i, 0)),
            pl.BlockSpec((1, 256, 128), lambda c, i: (c, 0, 0)),
        ],
        out_specs=pl.BlockSpec((BN, 128), lambda c, i: (c * (N // BN) + i, 0)),
        out_shape=jax.ShapeDtypeStruct((4 * N, 128), jnp.float32),
    )(h, Wt)

    z1pre = _s2(T, eC, srcx4, dstx4)

    out = pl.pallas_call(
        _pred_body,
        grid=(nE,),
        in_specs=[
            pl.BlockSpec((BE, 128), lambda i: (i, 0)),
            pl.BlockSpec((BE, 128), lambda i: (E // BE + i, 0)),
            pl.BlockSpec((256, 128), lambda i: (0, 0)),
            pl.BlockSpec((1, 128), lambda i: (0, 0)),
            pl.BlockSpec((128, 1), lambda i: (0, 0)),
            pl.BlockSpec((1, 1), lambda i: (0, 0)),
        ],
        out_specs=pl.BlockSpec((BE, 1), lambda i: (i, 0)),
        out_shape=jax.ShapeDtypeStruct((E, 1), jnp.float32),
    )(z1pre, z1pre, pp["l2"]["w"], _row2(pp["l2"]["b"]), pp["l3"]["w"],
      _row2(pp["l3"]["b"]))

    return out


# speedup vs baseline: 1.4636x; 1.0056x over previous
import functools

import jax
import jax.numpy as jnp
import numpy as np
from jax import lax
from jax.experimental import pallas as pl
from jax.experimental.pallas import tpu as pltpu
from jax.experimental.pallas import tpu_sc as plsc

N = 10000
E = 160000
HID = 256
NSUB = 16
E16 = E // NSUB
E32 = E // 2 // NSUB
B = 80
B0 = 40
NCH = E16 // B
NCH0 = E32 // B0
NP = 10240
RS = NP // NSUB

BN = 1000
BE = 2000

_EPS = 1e-5


def _pack_halves(v):
    lo = jax.lax.bitcast_convert_type(
        v[:, :128].astype(jnp.bfloat16).astype(jnp.float32), jnp.uint32)
    hi = jax.lax.bitcast_convert_type(
        v[:, 128:].astype(jnp.bfloat16).astype(jnp.float32), jnp.uint32)
    packed = (lo >> jnp.uint32(16)) | (hi & jnp.uint32(0xFFFF0000))
    return jax.lax.bitcast_convert_type(packed, jnp.int32)


def _ln(v, g, b):
    m = jnp.mean(v, axis=-1, keepdims=True)
    var = jnp.mean((v - m) ** 2, axis=-1, keepdims=True)
    return (v - m) / jnp.sqrt(var + _EPS) * g + b


def _node_enc_body(x_ref, w_ref, b_ref, g_ref, bb_ref, o_ref):
    v = jnp.maximum(jnp.dot(x_ref[...], w_ref[...]) + b_ref[...], 0.0)
    o_ref[...] = _ln(v, g_ref[...], bb_ref[...])


def _edge_proj_a_body(ea_ref, we_ref, be_ref, ge_ref, bbe_ref,
                      b1_ref, c1_ref, o1_ref):
    raw = ea_ref[...]
    enc = _ln(jnp.maximum(jnp.dot(raw, we_ref[...]) + be_ref[...], 0.0),
              ge_ref[...], bbe_ref[...])
    o1_ref[...] = _pack_halves(jnp.dot(enc, b1_ref[...]) + c1_ref[...])


def _edge_proj_b_body(ea_ref, we_ref, be_ref, ge_ref, bbe_ref,
                      b2_ref, c2_ref, cc_ref, ccb_ref, o2_ref, oc_ref):
    raw = ea_ref[...]
    enc = _ln(jnp.maximum(jnp.dot(raw, we_ref[...]) + be_ref[...], 0.0),
              ge_ref[...], bbe_ref[...])
    o2_ref[...] = _pack_halves(jnp.dot(enc, b2_ref[...]) + c2_ref[...])
    oc_ref[...] = _pack_halves(jnp.dot(raw, cc_ref[...]) + ccb_ref[...])


def _ha_body(h_ref, a_ref, o_ref):
    o_ref[...] = jnp.dot(h_ref[...], a_ref[0])


def _upd_body(h_ref, m0_ref, m1_ref, ca_ref, cb_ref, wu0_ref, wu1_ref,
              wu2_ref, bu_ref, g_ref, b_ref, o_ref):
    h = h_ref[...]
    cnt = ca_ref[:, 0:1] + cb_ref[:, 0:1]
    inv = 1.0 / jnp.maximum(cnt, 1.0)
    u = (jnp.dot(h, wu0_ref[...]) + jnp.dot(m0_ref[...] * inv, wu1_ref[...])
         + jnp.dot(m1_ref[...] * inv, wu2_ref[...]) + bu_ref[...])
    u = _ln(u, g_ref[...], b_ref[...])
    o_ref[...] = jnp.maximum(u + h, 0.0)


def _pred_body(zl_ref, zh_ref, w2_ref, b2_ref, w3_ref, b3_ref, o_ref):
    z = jnp.maximum(jnp.concatenate([zl_ref[...], zh_ref[...]], axis=1), 0.0)
    z = jnp.maximum(jnp.dot(z, w2_ref[...]) + b2_ref[...], 0.0)
    o_ref[...] = jax.nn.sigmoid(jnp.dot(z, w3_ref[...]) + b3_ref[...])


@functools.cache
def _sc_kernels():
  mesh = plsc.VectorSubcoreMesh(
      core_axis_name="c", subcore_axis_name="s", num_cores=2, num_subcores=NSUB)

  @functools.partial(
    pl.kernel,
    out_type=(jax.ShapeDtypeStruct((NP, 128), jnp.float32),
              jax.ShapeDtypeStruct((NP, 128), jnp.float32)),
    mesh=mesh,
    scratch_types=[
        pltpu.VMEM((B0,), jnp.int32),
        pltpu.VMEM((B0, 128), jnp.float32),
        pltpu.VMEM_SHARED((NP, 128), jnp.float32),
    ],
  )
  def _s0(dst, ones, zeros, out0, out1, dstv, onesv, shared):
    c = lax.axis_index("c")
    s = lax.axis_index("s")
    pltpu.sync_copy(zeros.at[pl.ds(s * RS, RS)], shared.at[pl.ds(s * RS, RS)])
    pltpu.sync_copy(ones, onesv)
    plsc.subcore_barrier()
    base = c * (E // 2) + s * E32

    def chunk(j, carry):
        pltpu.sync_copy(dst.at[pl.ds(base + j * B0, B0)], dstv)
        pltpu.sync_copy(onesv, shared.at[dstv], add=True)
        return carry

    lax.fori_loop(0, NCH0, chunk, 0)
    plsc.subcore_barrier()

    @pl.when(c == 0)
    def _():
        pltpu.sync_copy(shared.at[pl.ds(s * RS, RS)], out0.at[pl.ds(s * RS, RS)])

    @pl.when(c == 1)
    def _():
        pltpu.sync_copy(shared.at[pl.ds(s * RS, RS)], out1.at[pl.ds(s * RS, RS)])

  @functools.partial(
    pl.kernel,
    out_type=(jax.ShapeDtypeStruct((NP, 128), jnp.float32),
              jax.ShapeDtypeStruct((NP, 128), jnp.float32)),
    mesh=mesh,
    scratch_types=[
        pltpu.VMEM((2, B), jnp.int32),
        pltpu.VMEM((2, B), jnp.int32),
        pltpu.VMEM((2, B, 128), jnp.int32),
        pltpu.VMEM((2, B, 128), jnp.float32),
        pltpu.VMEM_SHARED((NP, 128), jnp.float32),
        pltpu.SemaphoreType.DMA,
        pltpu.SemaphoreType.DMA,
        pltpu.SemaphoreType.DMA,
        pltpu.SemaphoreType.DMA,
        pltpu.SemaphoreType.DMA,
        pltpu.SemaphoreType.DMA,
        pltpu.SemaphoreType.DMA,
        pltpu.SemaphoreType.DMA,
    ],
  )
  def _s1(hA, eaB, srcx4, dst3, zeros, out0, out1, srcv, dstv, ebuf, gbuf,
          shared, se0, se1, sg0, sg1, si0, si1, sd0, sd1):
    c = lax.axis_index("c")
    s = lax.axis_index("s")
    se = (se0, se1)
    sg = (sg0, sg1)
    si = (si0, si1)
    sd = (sd0, sd1)
    w = c * NSUB + s
    pltpu.sync_copy(zeros.at[pl.ds(s * RS, RS)], shared.at[pl.ds(s * RS, RS)])
    plsc.subcore_barrier()
    ebase = c * E + s * E16

    def src_copy(j, slot):
        return pltpu.make_async_copy(srcx4.at[w, j], srcv.at[slot], si[slot])

    def dst_copy(j, slot):
        return pltpu.make_async_copy(dst3.at[s, j], dstv.at[slot], sd[slot])

    sbase = s * E16
    shift = 16 * (1 - c)

    def eaB_copy(j, slot):
        return pltpu.make_async_copy(
            eaB.at[pl.ds(sbase + j * B, B)], ebuf.at[slot], se[slot])

    def gather_copy(slot):
        return pltpu.make_async_copy(
            hA.at[srcv.at[slot]], gbuf.at[slot], sg[slot])

    pltpu.sync_copy(srcx4.at[w, 0], srcv.at[0])
    pltpu.sync_copy(dst3.at[s, 0], dstv.at[0])
    src_copy(1, 1).start()
    dst_copy(1, 1).start()
    gather_copy(0).start()
    eaB_copy(0, 0).start()
    eaB_copy(1, 1).start()

    def body(j, slot):
        nslot = 1 - slot
        gather_copy(slot).wait()
        eaB_copy(j, slot).wait()

        def row(r, carry2):
            for k in range(128 // 16):
                sl = pl.ds(k * 16, 16)
                u = ebuf[slot, r, sl]
                v = jax.lax.bitcast_convert_type(
                    (u << shift) & jnp.int32(-65536), jnp.float32)
                gbuf[slot, r, sl] = jnp.maximum(gbuf[slot, r, sl] + v, 0.0)
            return carry2

        lax.fori_loop(0, B, row, 0)

        @pl.when(j + 1 < NCH)
        def _():
            src_copy(j + 1, nslot).wait()
            gather_copy(nslot).start()

        @pl.when(j > 0)
        def _():
            dst_copy(j, slot).wait()

        pltpu.sync_copy(gbuf.at[slot], shared.at[dstv.at[slot]], add=True)

        @pl.when(j + 2 < NCH)
        def _():
            src_copy(j + 2, slot).start()
            dst_copy(j + 2, slot).start()
            eaB_copy(j + 2, slot).start()

    def chunk(j, carry):
        @pl.when(j % 2 == 0)
        def _():
            body(j, 0)

        @pl.when(j % 2 == 1)
        def _():
            body(j, 1)

        return carry

    lax.fori_loop(0, NCH, chunk, 0)
    plsc.subcore_barrier()

    @pl.when(c == 0)
    def _():
        pltpu.sync_copy(shared.at[pl.ds(s * RS, RS)], out0.at[pl.ds(s * RS, RS)])

    @pl.when(c == 1)
    def _():
        pltpu.sync_copy(shared.at[pl.ds(s * RS, RS)], out1.at[pl.ds(s * RS, RS)])

  @functools.partial(
    pl.kernel,
    out_type=jax.ShapeDtypeStruct((2 * E, 128), jnp.float32),
    mesh=mesh,
    scratch_types=[
        pltpu.VMEM((NCH, B), jnp.int32),
        pltpu.VMEM((NCH, B), jnp.int32),
        pltpu.VMEM((2, B, 128), jnp.int32),
        pltpu.VMEM((2, B, 128), jnp.float32),
        pltpu.VMEM((2, B, 128), jnp.float32),
        pltpu.VMEM((2, B, 128), jnp.float32),
        pltpu.SemaphoreType.DMA,
        pltpu.SemaphoreType.DMA,
        pltpu.SemaphoreType.DMA,
        pltpu.SemaphoreType.DMA,
        pltpu.SemaphoreType.DMA,
        pltpu.SemaphoreType.DMA,
        pltpu.SemaphoreType.DMA,
        pltpu.SemaphoreType.DMA,
    ],
  )
  def _s2(T, eC, srcx4, dstx4, out, src_t, dst_t, ebuf, g1, g2, obuf,
          se0, se1, sa0, sa1, sb0, sb1, sw0, sw1):
    c = lax.axis_index("c")
    s = lax.axis_index("s")
    se = (se0, se1)
    sa = (sa0, sa1)
    sb = (sb0, sb1)
    sw = (sw0, sw1)
    pltpu.sync_copy(srcx4.at[c * NSUB + s], src_t)
    pltpu.sync_copy(dstx4.at[c * NSUB + s], dst_t)
    ebase = c * E + s * E16

    sbase = s * E16
    shift = 16 * (1 - c)

    def eC_copy(j, slot):
        return pltpu.make_async_copy(
            eC.at[pl.ds(sbase + j * B, B)], ebuf.at[slot], se[slot])

    def ga_copy(j, slot):
        return pltpu.make_async_copy(T.at[src_t.at[j]], g1.at[slot], sa[slot])

    def gb_copy(j, slot):
        return pltpu.make_async_copy(T.at[dst_t.at[j]], g2.at[slot], sb[slot])

    def out_copy(j, slot):
        return pltpu.make_async_copy(
            obuf.at[slot], out.at[pl.ds(ebase + j * B, B)], sw[slot])

    for slot in (0, 1):
        eC_copy(slot, slot).start()
        ga_copy(slot, slot).start()
        gb_copy(slot, slot).start()

    def body(j, slot):
        eC_copy(j, slot).wait()
        ga_copy(j, slot).wait()
        gb_copy(j, slot).wait()

        @pl.when(j >= 2)
        def _():
            out_copy(j, slot).wait()

        def row(r, carry2):
            for k in range(128 // 16):
                sl = pl.ds(k * 16, 16)
                u = ebuf[slot, r, sl]
                v = jax.lax.bitcast_convert_type(
                    (u << shift) & jnp.int32(-65536), jnp.float32)
                obuf[slot, r, sl] = g1[slot, r, sl] + g2[slot, r, sl] + v
            return carry2

        lax.fori_loop(0, B, row, 0)
        out_copy(j, slot).start()

        @pl.when(j + 2 < NCH)
        def _():
            eC_copy(j + 2, slot).start()
            ga_copy(j + 2, slot).start()
            gb_copy(j + 2, slot).start()

    def chunk(j, carry):
        @pl.when(j % 2 == 0)
        def _():
            body(j, 0)

        @pl.when(j % 2 == 1)
        def _():
            body(j, 1)

        return carry

    lax.fori_loop(0, NCH, chunk, 0)
    out_copy(NCH - 2, (NCH - 2) % 2).wait()
    out_copy(NCH - 1, (NCH - 1) % 2).wait()

  return _s0, _s1, _s2


def _row2(m):
    return m.reshape(1, -1)


def _split_stack(w):
    return w.reshape(w.shape[0], 2, 128).transpose(1, 0, 2)


def kernel(x, edge_index, edge_attr, params):
    src = edge_index[0]
    dst = edge_index[1]
    nE = E // BE
    nN = N // BN

    pn = params["node_enc"]
    pe = params["edge_enc"]
    pp = params["pred"]

    h = pl.pallas_call(
        _node_enc_body,
        grid=(nN,),
        in_specs=[
            pl.BlockSpec((BN, 128), lambda i: (i, 0)),
            pl.BlockSpec((128, 256), lambda i: (0, 0)),
            pl.BlockSpec((1, 256), lambda i: (0, 0)),
            pl.BlockSpec((1, 256), lambda i: (0, 0)),
            pl.BlockSpec((1, 256), lambda i: (0, 0)),
        ],
        out_specs=pl.BlockSpec((BN, 256), lambda i: (i, 0)),
        out_shape=jax.ShapeDtypeStruct((N, 256), jnp.float32),
    )(x, pn["lin"]["w"], _row2(pn["lin"]["b"]), _row2(pn["ln_g"]),
      _row2(pn["ln_b"]))

    Bst = [cp["msg"]["w"][256:] for cp in params["convs"]]
    bst = [_row2(cp["msg"]["b"]) for cp in params["convs"]]
    W1 = pp["l1"]["w"]
    Cst = W1[512:]
    cbs = _row2(pp["l1"]["b"])

    _enc_specs = [
        pl.BlockSpec((BE, 16), lambda i: (i, 0)),
        pl.BlockSpec((16, 256), lambda i: (0, 0)),
        pl.BlockSpec((1, 256), lambda i: (0, 0)),
        pl.BlockSpec((1, 256), lambda i: (0, 0)),
        pl.BlockSpec((1, 256), lambda i: (0, 0)),
    ]
    _enc_args = (edge_attr, pe["lin"]["w"], _row2(pe["lin"]["b"]),
                 _row2(pe["ln_g"]), _row2(pe["ln_b"]))
    _eout = pl.BlockSpec((BE, 128), lambda i: (i, 0))

    srcx4 = jnp.concatenate([src, src + N]).reshape(2 * NSUB, NCH, B)
    dstx4 = jnp.concatenate([dst + 2 * N, dst + 3 * N]).reshape(2 * NSUB, NCH, B)
    dst3 = dst.reshape(NSUB, NCH, B)
    zeros = jnp.zeros((NP, 128), jnp.float32)
    ones = jnp.ones((B0, 128), jnp.float32)
    _s0, _s1, _s2 = _sc_kernels()

    cnta, cntb = _s0(dst, ones, zeros)

    eaB1 = pl.pallas_call(
        _edge_proj_a_body,
        grid=(nE,),
        in_specs=_enc_specs + [
            pl.BlockSpec((256, 256), lambda i: (0, 0)),
            pl.BlockSpec((1, 256), lambda i: (0, 0)),
        ],
        out_specs=_eout,
        out_shape=jax.ShapeDtypeStruct((E, 128), jnp.int32),
    )(*_enc_args, Bst[0], bst[0])

    eaB2 = eC = None
    for li, cp in enumerate(params["convs"]):
        Ast = _split_stack(cp["msg"]["w"][:256])
        hA = pl.pallas_call(
            _ha_body,
            grid=(2, nN),
            in_specs=[
                pl.BlockSpec((BN, 256), lambda c, i: (i, 0)),
                pl.BlockSpec((1, 256, 128), lambda c, i: (c, 0, 0)),
            ],
            out_specs=pl.BlockSpec((BN, 128), lambda c, i: (c * (N // BN) + i, 0)),
            out_shape=jax.ShapeDtypeStruct((2 * N, 128), jnp.float32),
        )(h, Ast)

        msum0, msum1 = _s1(hA, eaB1 if li == 0 else eaB2, srcx4, dst3, zeros)

        if li == 0:
            eaB2, eC = pl.pallas_call(
                _edge_proj_b_body,
                grid=(nE,),
                in_specs=_enc_specs + [
                    pl.BlockSpec((256, 256), lambda i: (0, 0)),
                    pl.BlockSpec((1, 256), lambda i: (0, 0)),
                    pl.BlockSpec((16, 256), lambda i: (0, 0)),
                    pl.BlockSpec((1, 256), lambda i: (0, 0)),
                ],
                out_specs=[_eout, _eout],
                out_shape=[
                    jax.ShapeDtypeStruct((E, 128), jnp.int32),
                    jax.ShapeDtypeStruct((E, 128), jnp.int32),
                ],
            )(*_enc_args, Bst[1], bst[1], Cst, cbs)

        Wu = cp["upd"]["w"]
        h = pl.pallas_call(
            _upd_body,
            grid=(nN,),
            in_specs=[
                pl.BlockSpec((BN, 256), lambda i: (i, 0)),
                pl.BlockSpec((BN, 128), lambda i: (i, 0)),
                pl.BlockSpec((BN, 128), lambda i: (i, 0)),
                pl.BlockSpec((BN, 128), lambda i: (i, 0)),
                pl.BlockSpec((BN, 128), lambda i: (i, 0)),
                pl.BlockSpec((256, 256), lambda i: (0, 0)),
                pl.BlockSpec((128, 256), lambda i: (0, 0)),
                pl.BlockSpec((128, 256), lambda i: (0, 0)),
                pl.BlockSpec((1, 256), lambda i: (0, 0)),
                pl.BlockSpec((1, 256), lambda i: (0, 0)),
                pl.BlockSpec((1, 256), lambda i: (0, 0)),
            ],
            out_specs=pl.BlockSpec((BN, 256), lambda i: (i, 0)),
            out_shape=jax.ShapeDtypeStruct((N, 256), jnp.float32),
        )(h, msum0, msum1, cnta, cntb, Wu[:256], Wu[256:384], Wu[384:512],
          _row2(cp["upd"]["b"]), _row2(cp["ln_g"]), _row2(cp["ln_b"]))

    Wt = jnp.stack([
        W1[:256, :128], W1[:256, 128:256], W1[256:512, :128], W1[256:512, 128:256],
    ])
    T = pl.pallas_call(
        _ha_body,
        grid=(4, nN),
        in_specs=[
            pl.BlockSpec((BN, 256), lambda c, i: (i, 0)),
            pl.BlockSpec((1, 256, 128), lambda c, i: (c, 0, 0)),
        ],
        out_specs=pl.BlockSpec((BN, 128), lambda c, i: (c * (N // BN) + i, 0)),
        out_shape=jax.ShapeDtypeStruct((4 * N, 128), jnp.float32),
    )(h, Wt)

    z1pre = _s2(T, eC, srcx4, dstx4)

    out = pl.pallas_call(
        _pred_body,
        grid=(nE,),
        in_specs=[
            pl.BlockSpec((BE, 128), lambda i: (i, 0)),
            pl.BlockSpec((BE, 128), lambda i: (E // BE + i, 0)),
            pl.BlockSpec((256, 128), lambda i: (0, 0)),
            pl.BlockSpec((1, 128), lambda i: (0, 0)),
            pl.BlockSpec((128, 1), lambda i: (0, 0)),
            pl.BlockSpec((1, 1), lambda i: (0, 0)),
        ],
        out_specs=pl.BlockSpec((BE, 1), lambda i: (i, 0)),
        out_shape=jax.ShapeDtypeStruct((E, 1), jnp.float32),
    )(z1pre, z1pre, pp["l2"]["w"], _row2(pp["l2"]["b"]), pp["l3"]["w"],
      _row2(pp["l3"]["b"]))

    return out
